# baseline (speedup 1.0000x reference)
import functools

import jax
import jax.numpy as jnp
from jax import lax
from jax.experimental import pallas as pl
from jax.experimental.pallas import tpu as pltpu
from jax.experimental.pallas import tpu_sc as plsc

N = 2048
E = 65536
DE = 16
NC = 2
NS = 16
NW = NC * NS
EPT = E // NW
CH = 128
NCH = EPT // CH
NPT = N // NS


def _gather_body(x_hbm, src_hbm, xs_hbm, idx_v, rows_v, sem):
    c = lax.axis_index("c")
    s = lax.axis_index("s")
    wid = s * NC + c
    pltpu.sync_copy(src_hbm.at[wid], idx_v)
    base = wid * EPT
    for j in range(NCH):
        pltpu.async_copy(x_hbm.at[idx_v.at[j]], rows_v, sem).wait()
        pltpu.sync_copy(rows_v, xs_hbm.at[pl.ds(base + j * CH, CH)])


_SC_PARAMS = pltpu.CompilerParams(use_tc_tiling_on_sc=False)


def _sc_gather(x, src3d, in_c):
    mesh = plsc.VectorSubcoreMesh(core_axis_name="c", subcore_axis_name="s")
    k = pl.kernel(
        _gather_body,
        mesh=mesh,
        compiler_params=_SC_PARAMS,
        out_type=jax.ShapeDtypeStruct((E, in_c), jnp.float32),
        scratch_types=[
            pltpu.VMEM((NCH, CH), jnp.int32),
            pltpu.VMEM((CH, in_c), jnp.float32),
            pltpu.SemaphoreType.DMA,
        ],
    )
    return k(x, src3d)


def _scatter_cnt_body(msg_hbm, dst_hbm, zeros_hbm, zc_hbm, ones_hbm,
                      agg_hbm, cnt_hbm, idx_v, rows_v, ones_v, agg_sh, cnt_sh):
    c = lax.axis_index("c")
    s = lax.axis_index("s")
    wid = s * NC + c
    r0 = s * NPT
    pltpu.sync_copy(zeros_hbm.at[pl.ds(r0, NPT)], agg_sh.at[pl.ds(r0, NPT)])
    pltpu.sync_copy(zc_hbm.at[pl.ds(r0, NPT)], cnt_sh.at[pl.ds(r0, NPT)])
    pltpu.sync_copy(ones_hbm, ones_v)
    pltpu.sync_copy(dst_hbm.at[wid], idx_v)
    plsc.subcore_barrier()
    base = wid * EPT
    for j in range(NCH):
        pltpu.sync_copy(msg_hbm.at[pl.ds(base + j * CH, CH)], rows_v)
        pltpu.sync_copy(rows_v, agg_sh.at[idx_v.at[j]], add=True)
        pltpu.sync_copy(ones_v, cnt_sh.at[idx_v.at[j]], add=True)
    plsc.subcore_barrier()
    pltpu.sync_copy(agg_sh.at[pl.ds(r0, NPT)], agg_hbm.at[c, pl.ds(r0, NPT)])
    pltpu.sync_copy(cnt_sh.at[pl.ds(r0, NPT)], cnt_hbm.at[c, pl.ds(r0, NPT)])


def _scatter_body(msg_hbm, dst_hbm, zeros_hbm, agg_hbm, idx_v, rows_v, agg_sh):
    c = lax.axis_index("c")
    s = lax.axis_index("s")
    wid = s * NC + c
    r0 = s * NPT
    pltpu.sync_copy(zeros_hbm.at[pl.ds(r0, NPT)], agg_sh.at[pl.ds(r0, NPT)])
    pltpu.sync_copy(dst_hbm.at[wid], idx_v)
    plsc.subcore_barrier()
    base = wid * EPT
    for j in range(NCH):
        pltpu.sync_copy(msg_hbm.at[pl.ds(base + j * CH, CH)], rows_v)
        pltpu.sync_copy(rows_v, agg_sh.at[idx_v.at[j]], add=True)
    plsc.subcore_barrier()
    pltpu.sync_copy(agg_sh.at[pl.ds(r0, NPT)], agg_hbm.at[c, pl.ds(r0, NPT)])


def _sc_scatter(msg, dst3d, out_c, with_cnt):
    mesh = plsc.VectorSubcoreMesh(core_axis_name="c", subcore_axis_name="s")
    zeros = jnp.zeros((N, out_c), jnp.float32)
    if with_cnt:
        zc = jnp.zeros((N, 16), jnp.float32)
        ones = jnp.ones((CH, 16), jnp.float32)
        k = pl.kernel(
            _scatter_cnt_body,
            mesh=mesh,
            compiler_params=_SC_PARAMS,
            out_type=(jax.ShapeDtypeStruct((NC, N, out_c), jnp.float32),
                      jax.ShapeDtypeStruct((NC, N, 16), jnp.float32)),
            scratch_types=[
                pltpu.VMEM((NCH, CH), jnp.int32),
                pltpu.VMEM((CH, out_c), jnp.float32),
                pltpu.VMEM((CH, 16), jnp.float32),
                pltpu.VMEM_SHARED((N, out_c), jnp.float32),
                pltpu.VMEM_SHARED((N, 16), jnp.float32),
            ],
        )
        return k(msg, dst3d, zeros, zc, ones)
    k = pl.kernel(
        _scatter_body,
        mesh=mesh,
        compiler_params=_SC_PARAMS,
        out_type=jax.ShapeDtypeStruct((NC, N, out_c), jnp.float32),
        scratch_types=[
            pltpu.VMEM((NCH, CH), jnp.int32),
            pltpu.VMEM((CH, out_c), jnp.float32),
            pltpu.VMEM_SHARED((N, out_c), jnp.float32),
        ],
    )
    return k(msg, dst3d, zeros)


def _edge_body(in_c, out_c):
    def body(ea_ref, xs_ref, w_ref, b_ref, msg_ref):
        h = jnp.dot(ea_ref[...], w_ref[...], preferred_element_type=jnp.float32)
        h = jnp.maximum(h + b_ref[...], 0.0)
        xs = xs_ref[...]
        acc = xs[:, 0:1] * h[:, 0:out_c]
        for i in range(1, in_c):
            acc = acc + xs[:, i:i + 1] * h[:, i * out_c:(i + 1) * out_c]
        msg_ref[...] = acc
    return body


def _edge_tc(ea, xs, w, b, in_c, out_c, eb=512):
    return pl.pallas_call(
        _edge_body(in_c, out_c),
        grid=(E // eb,),
        in_specs=[
            pl.BlockSpec((eb, DE), lambda i: (i, 0)),
            pl.BlockSpec((eb, in_c), lambda i: (i, 0)),
            pl.BlockSpec((DE, in_c * out_c), lambda i: (0, 0)),
            pl.BlockSpec((1, in_c * out_c), lambda i: (0, 0)),
        ],
        out_specs=pl.BlockSpec((eb, out_c), lambda i: (i, 0)),
        out_shape=jax.ShapeDtypeStruct((E, out_c), jnp.float32),
    )(ea, xs, w, b)


def _update_body(x_ref, agg_ref, cnt_ref, root_ref, b_ref, h_ref):
    agg3 = agg_ref[...]
    agg = agg3[0] + agg3[1]
    cnt3 = cnt_ref[...]
    cnt = cnt3[0, :, 0:1] + cnt3[1, :, 0:1]
    inv = 1.0 / jnp.maximum(cnt, 1.0)
    h = jnp.dot(x_ref[...], root_ref[...], preferred_element_type=jnp.float32)
    h_ref[...] = jnp.maximum(h + agg * inv + b_ref[...], 0.0)


def _update_tc(x, agg, cnt, root, b, in_c, out_c):
    return pl.pallas_call(
        _update_body,
        out_shape=jax.ShapeDtypeStruct((N, out_c), jnp.float32),
    )(x, agg, cnt, root, b)


def _cbt_body(d):
    def body(ha_ref, hb_ref, o_ref):
        ha = ha_ref[...]
        hbt = hb_ref[...].T
        acc = jnp.abs(ha[:, 0:1] - hbt[0:1, :])
        for k in range(1, d):
            acc = acc + jnp.abs(ha[:, k:k + 1] - hbt[k:k + 1, :])
        o_ref[...] = acc
    return body


def _cbt_tc(h, d, ba=256, bb=256):
    return pl.pallas_call(
        _cbt_body(d),
        grid=(N // ba, N // bb),
        in_specs=[
            pl.BlockSpec((ba, d), lambda i, j: (i, 0)),
            pl.BlockSpec((bb, d), lambda i, j: (j, 0)),
        ],
        out_specs=pl.BlockSpec((ba, bb), lambda i, j: (i, j)),
        out_shape=jax.ShapeDtypeStruct((N, N), jnp.float32),
    )(h, h)


def kernel(x, edge_attr, edge_index, W_nn1, b_nn1, root1, bias1,
           W_nn2, b_nn2, root2, bias2, W_nn3, b_nn3, root3, bias3):
    src3d = edge_index[0].reshape(NW, NCH, CH)
    dst3d = edge_index[1].reshape(NW, NCH, CH)
    layers = [
        (W_nn1, b_nn1, root1, bias1, 32, 64),
        (W_nn2, b_nn2, root2, bias2, 64, 32),
        (W_nn3, b_nn3, root3, bias3, 32, 16),
    ]
    h = x
    cnt = None
    for li, (w, b, root, bias, in_c, out_c) in enumerate(layers):
        xs = _sc_gather(h, src3d, in_c)
        msg = _edge_tc(edge_attr, xs, w, b.reshape(1, -1), in_c, out_c)
        if li == 0:
            agg, cnt = _sc_scatter(msg, dst3d, out_c, True)
        else:
            agg = _sc_scatter(msg, dst3d, out_c, False)
        h = _update_tc(h, agg, cnt, root, bias.reshape(1, -1), in_c, out_c)
    return _cbt_tc(h, 16)


# speedup vs baseline: 3.1022x; 3.1022x over previous
import functools

import jax
import jax.numpy as jnp
from jax import lax
from jax.experimental import pallas as pl
from jax.experimental.pallas import tpu as pltpu
from jax.experimental.pallas import tpu_sc as plsc

N = 2048
E = 65536
DE = 16
NC = 2
NS = 16
NW = NC * NS
EPT = E // NW
CH = 128
NCH = EPT // CH
NPT = N // NS


def _gather_body(x_hbm, src_hbm, xs_hbm, idx_v, rows_v, sem):
    c = lax.axis_index("c")
    s = lax.axis_index("s")
    wid = s * NC + c
    pltpu.sync_copy(src_hbm.at[wid], idx_v)
    base = wid * EPT
    for j in range(NCH):
        pltpu.async_copy(x_hbm.at[idx_v.at[j]], rows_v, sem).wait()
        pltpu.sync_copy(rows_v, xs_hbm.at[pl.ds(base + j * CH, CH)])


_SC_PARAMS = pltpu.CompilerParams(use_tc_tiling_on_sc=False)


def _sc_gather(x, src3d, in_c):
    mesh = plsc.VectorSubcoreMesh(core_axis_name="c", subcore_axis_name="s")
    k = pl.kernel(
        _gather_body,
        mesh=mesh,
        compiler_params=_SC_PARAMS,
        out_type=jax.ShapeDtypeStruct((E, in_c), jnp.float32),
        scratch_types=[
            pltpu.VMEM((NCH, CH), jnp.int32),
            pltpu.VMEM((CH, in_c), jnp.float32),
            pltpu.SemaphoreType.DMA,
        ],
    )
    return k(x, src3d)


def _scatter_cnt_body(msg_hbm, dst_hbm, zeros_hbm, zc_hbm, ones_hbm,
                      agg_hbm, cnt_hbm, idx_v, rows_v, ones_v, agg_sh, cnt_sh):
    c = lax.axis_index("c")
    s = lax.axis_index("s")
    wid = s * NC + c
    r0 = s * NPT
    pltpu.sync_copy(zeros_hbm.at[pl.ds(r0, NPT)], agg_sh.at[pl.ds(r0, NPT)])
    pltpu.sync_copy(zc_hbm.at[pl.ds(r0, NPT)], cnt_sh.at[pl.ds(r0, NPT)])
    pltpu.sync_copy(ones_hbm, ones_v)
    pltpu.sync_copy(dst_hbm.at[wid], idx_v)
    plsc.subcore_barrier()
    base = wid * EPT
    for j in range(NCH):
        pltpu.sync_copy(msg_hbm.at[pl.ds(base + j * CH, CH)], rows_v)
        pltpu.sync_copy(rows_v, agg_sh.at[idx_v.at[j]], add=True)
        pltpu.sync_copy(ones_v, cnt_sh.at[idx_v.at[j]], add=True)
    plsc.subcore_barrier()
    pltpu.sync_copy(agg_sh.at[pl.ds(r0, NPT)], agg_hbm.at[c, pl.ds(r0, NPT)])
    pltpu.sync_copy(cnt_sh.at[pl.ds(r0, NPT)], cnt_hbm.at[c, pl.ds(r0, NPT)])


def _scatter_body(msg_hbm, dst_hbm, zeros_hbm, agg_hbm, idx_v, rows_v, agg_sh):
    c = lax.axis_index("c")
    s = lax.axis_index("s")
    wid = s * NC + c
    r0 = s * NPT
    pltpu.sync_copy(zeros_hbm.at[pl.ds(r0, NPT)], agg_sh.at[pl.ds(r0, NPT)])
    pltpu.sync_copy(dst_hbm.at[wid], idx_v)
    plsc.subcore_barrier()
    base = wid * EPT
    for j in range(NCH):
        pltpu.sync_copy(msg_hbm.at[pl.ds(base + j * CH, CH)], rows_v)
        pltpu.sync_copy(rows_v, agg_sh.at[idx_v.at[j]], add=True)
    plsc.subcore_barrier()
    pltpu.sync_copy(agg_sh.at[pl.ds(r0, NPT)], agg_hbm.at[c, pl.ds(r0, NPT)])


def _sc_scatter(msg, dst3d, out_c, with_cnt):
    mesh = plsc.VectorSubcoreMesh(core_axis_name="c", subcore_axis_name="s")
    zeros = jnp.zeros((N, out_c), jnp.float32)
    if with_cnt:
        zc = jnp.zeros((N, 16), jnp.float32)
        ones = jnp.ones((CH, 16), jnp.float32)
        k = pl.kernel(
            _scatter_cnt_body,
            mesh=mesh,
            compiler_params=_SC_PARAMS,
            out_type=(jax.ShapeDtypeStruct((NC, N, out_c), jnp.float32),
                      jax.ShapeDtypeStruct((NC, N, 16), jnp.float32)),
            scratch_types=[
                pltpu.VMEM((NCH, CH), jnp.int32),
                pltpu.VMEM((CH, out_c), jnp.float32),
                pltpu.VMEM((CH, 16), jnp.float32),
                pltpu.VMEM_SHARED((N, out_c), jnp.float32),
                pltpu.VMEM_SHARED((N, 16), jnp.float32),
            ],
        )
        return k(msg, dst3d, zeros, zc, ones)
    k = pl.kernel(
        _scatter_body,
        mesh=mesh,
        compiler_params=_SC_PARAMS,
        out_type=jax.ShapeDtypeStruct((NC, N, out_c), jnp.float32),
        scratch_types=[
            pltpu.VMEM((NCH, CH), jnp.int32),
            pltpu.VMEM((CH, out_c), jnp.float32),
            pltpu.VMEM_SHARED((N, out_c), jnp.float32),
        ],
    )
    return k(msg, dst3d, zeros)


def _edge_body(in_c, out_c):
    def body(ea_ref, xs_ref, wt_ref, bt_ref, msg_ref):
        eat = ea_ref[...].T
        h = jnp.dot(wt_ref[...], eat, preferred_element_type=jnp.float32)
        h = jnp.maximum(h + bt_ref[...], 0.0)
        xst = xs_ref[...].T
        acc = xst[0:1, :] * h[0:out_c, :]
        for i in range(1, in_c):
            acc = acc + xst[i:i + 1, :] * h[i * out_c:(i + 1) * out_c, :]
        msg_ref[...] = acc.T
    return body


def _edge_tc(ea, xs, wt, bt, in_c, out_c, eb=512):
    return pl.pallas_call(
        _edge_body(in_c, out_c),
        grid=(E // eb,),
        in_specs=[
            pl.BlockSpec((eb, DE), lambda i: (i, 0)),
            pl.BlockSpec((eb, in_c), lambda i: (i, 0)),
            pl.BlockSpec((in_c * out_c, DE), lambda i: (0, 0)),
            pl.BlockSpec((in_c * out_c, 1), lambda i: (0, 0)),
        ],
        out_specs=pl.BlockSpec((eb, out_c), lambda i: (i, 0)),
        out_shape=jax.ShapeDtypeStruct((E, out_c), jnp.float32),
    )(ea, xs, wt, bt)


def _update_body(x_ref, agg_ref, cnt_ref, root_ref, b_ref, h_ref):
    agg3 = agg_ref[...]
    agg = agg3[0] + agg3[1]
    cnt3 = cnt_ref[...]
    cnt = cnt3[0, :, 0:1] + cnt3[1, :, 0:1]
    inv = 1.0 / jnp.maximum(cnt, 1.0)
    h = jnp.dot(x_ref[...], root_ref[...], preferred_element_type=jnp.float32)
    h_ref[...] = jnp.maximum(h + agg * inv + b_ref[...], 0.0)


def _update_tc(x, agg, cnt, root, b, in_c, out_c):
    return pl.pallas_call(
        _update_body,
        out_shape=jax.ShapeDtypeStruct((N, out_c), jnp.float32),
    )(x, agg, cnt, root, b)


def _cbt_body(d):
    def body(ha_ref, hb_ref, o_ref):
        ha = ha_ref[...]
        hbt = hb_ref[...].T
        acc = jnp.abs(ha[:, 0:1] - hbt[0:1, :])
        for k in range(1, d):
            acc = acc + jnp.abs(ha[:, k:k + 1] - hbt[k:k + 1, :])
        o_ref[...] = acc
    return body


def _cbt_tc(h, d, ba=256, bb=256):
    return pl.pallas_call(
        _cbt_body(d),
        grid=(N // ba, N // bb),
        in_specs=[
            pl.BlockSpec((ba, d), lambda i, j: (i, 0)),
            pl.BlockSpec((bb, d), lambda i, j: (j, 0)),
        ],
        out_specs=pl.BlockSpec((ba, bb), lambda i, j: (i, j)),
        out_shape=jax.ShapeDtypeStruct((N, N), jnp.float32),
    )(h, h)


def kernel(x, edge_attr, edge_index, W_nn1, b_nn1, root1, bias1,
           W_nn2, b_nn2, root2, bias2, W_nn3, b_nn3, root3, bias3):
    src3d = edge_index[0].reshape(NW, NCH, CH)
    dst3d = edge_index[1].reshape(NW, NCH, CH)
    layers = [
        (W_nn1, b_nn1, root1, bias1, 32, 64),
        (W_nn2, b_nn2, root2, bias2, 64, 32),
        (W_nn3, b_nn3, root3, bias3, 32, 16),
    ]
    h = x
    cnt = None
    for li, (w, b, root, bias, in_c, out_c) in enumerate(layers):
        xs = _sc_gather(h, src3d, in_c)
        msg = _edge_tc(edge_attr, xs, w.T, b.reshape(-1, 1), in_c, out_c)
        if li == 0:
            agg, cnt = _sc_scatter(msg, dst3d, out_c, True)
        else:
            agg = _sc_scatter(msg, dst3d, out_c, False)
        h = _update_tc(h, agg, cnt, root, bias.reshape(1, -1), in_c, out_c)
    return _cbt_tc(h, 16)


# speedup vs baseline: 3.3115x; 1.0675x over previous
import functools

import jax
import jax.numpy as jnp
from jax import lax
from jax.experimental import pallas as pl
from jax.experimental.pallas import tpu as pltpu
from jax.experimental.pallas import tpu_sc as plsc

N = 2048
E = 65536
DE = 16
NC = 2
NS = 16
NW = NC * NS
EPT = E // NW
CH = 128
NCH = EPT // CH
NPT = N // NS


def _gather_body(nbuf):
    def body(x_hbm, src_hbm, xs_hbm, idx_v, rows_v, gsem, wsem):
        c = lax.axis_index("c")
        s = lax.axis_index("s")
        wid = s * NC + c
        pltpu.sync_copy(src_hbm.at[wid], idx_v)
        base = wid * EPT
        for w0 in range(0, NCH, nbuf):
            gs = [pltpu.async_copy(x_hbm.at[idx_v.at[w0 + t]], rows_v.at[t], gsem)
                  for t in range(nbuf)]
            for cp in gs:
                cp.wait()
            ws = [pltpu.async_copy(rows_v.at[t],
                                   xs_hbm.at[pl.ds(base + (w0 + t) * CH, CH)], wsem)
                  for t in range(nbuf)]
            for cp in ws:
                cp.wait()
    return body


_SC_PARAMS = pltpu.CompilerParams(use_tc_tiling_on_sc=False)


def _sc_gather(x, src3d, in_c):
    mesh = plsc.VectorSubcoreMesh(core_axis_name="c", subcore_axis_name="s")
    nbuf = 16 if in_c <= 32 else 8
    k = pl.kernel(
        _gather_body(nbuf),
        mesh=mesh,
        compiler_params=_SC_PARAMS,
        out_type=jax.ShapeDtypeStruct((E, in_c), jnp.float32),
        scratch_types=[
            pltpu.VMEM((NCH, CH), jnp.int32),
            pltpu.VMEM((nbuf, CH, in_c), jnp.float32),
            pltpu.SemaphoreType.DMA,
            pltpu.SemaphoreType.DMA,
        ],
    )
    return k(x, src3d)


def _scatter_cnt_body(nbuf):
    def body(msg_hbm, dst_hbm, zeros_hbm, zc_hbm, ones_hbm,
             agg_hbm, cnt_hbm, idx_v, rows_v, ones_v, agg_sh, cnt_sh,
             lsem, ssem):
        c = lax.axis_index("c")
        s = lax.axis_index("s")
        wid = s * NC + c
        r0 = s * NPT
        pltpu.sync_copy(zeros_hbm.at[pl.ds(r0, NPT)], agg_sh.at[pl.ds(r0, NPT)])
        pltpu.sync_copy(zc_hbm.at[pl.ds(r0, NPT)], cnt_sh.at[pl.ds(r0, NPT)])
        pltpu.sync_copy(ones_hbm, ones_v)
        pltpu.sync_copy(dst_hbm.at[wid], idx_v)
        plsc.subcore_barrier()
        base = wid * EPT
        for w0 in range(0, NCH, nbuf):
            ld = [pltpu.async_copy(msg_hbm.at[pl.ds(base + (w0 + t) * CH, CH)],
                                   rows_v.at[t], lsem) for t in range(nbuf)]
            for cp in ld:
                cp.wait()
            sc = [pltpu.async_copy(rows_v.at[t], agg_sh.at[idx_v.at[w0 + t]],
                                   ssem, add=True) for t in range(nbuf)]
            sc += [pltpu.async_copy(ones_v, cnt_sh.at[idx_v.at[w0 + t]],
                                    ssem, add=True) for t in range(nbuf)]
            for cp in sc:
                cp.wait()
        plsc.subcore_barrier()
        pltpu.sync_copy(agg_sh.at[pl.ds(r0, NPT)], agg_hbm.at[c, pl.ds(r0, NPT)])
        pltpu.sync_copy(cnt_sh.at[pl.ds(r0, NPT)], cnt_hbm.at[c, pl.ds(r0, NPT)])
    return body


def _scatter_body(nbuf):
    def body(msg_hbm, dst_hbm, zeros_hbm, agg_hbm, idx_v, rows_v, agg_sh,
             lsem, ssem):
        c = lax.axis_index("c")
        s = lax.axis_index("s")
        wid = s * NC + c
        r0 = s * NPT
        pltpu.sync_copy(zeros_hbm.at[pl.ds(r0, NPT)], agg_sh.at[pl.ds(r0, NPT)])
        pltpu.sync_copy(dst_hbm.at[wid], idx_v)
        plsc.subcore_barrier()
        base = wid * EPT
        for w0 in range(0, NCH, nbuf):
            ld = [pltpu.async_copy(msg_hbm.at[pl.ds(base + (w0 + t) * CH, CH)],
                                   rows_v.at[t], lsem) for t in range(nbuf)]
            for cp in ld:
                cp.wait()
            sc = [pltpu.async_copy(rows_v.at[t], agg_sh.at[idx_v.at[w0 + t]],
                                   ssem, add=True) for t in range(nbuf)]
            for cp in sc:
                cp.wait()
        plsc.subcore_barrier()
        pltpu.sync_copy(agg_sh.at[pl.ds(r0, NPT)], agg_hbm.at[c, pl.ds(r0, NPT)])
    return body


def _sc_scatter(msg, dst3d, out_c, with_cnt):
    mesh = plsc.VectorSubcoreMesh(core_axis_name="c", subcore_axis_name="s")
    zeros = jnp.zeros((N, out_c), jnp.float32)
    nbuf = 16 if out_c <= 32 else 8
    if with_cnt:
        zc = jnp.zeros((N, 16), jnp.float32)
        ones = jnp.ones((CH, 16), jnp.float32)
        k = pl.kernel(
            _scatter_cnt_body(nbuf),
            mesh=mesh,
            compiler_params=_SC_PARAMS,
            out_type=(jax.ShapeDtypeStruct((NC, N, out_c), jnp.float32),
                      jax.ShapeDtypeStruct((NC, N, 16), jnp.float32)),
            scratch_types=[
                pltpu.VMEM((NCH, CH), jnp.int32),
                pltpu.VMEM((nbuf, CH, out_c), jnp.float32),
                pltpu.VMEM((CH, 16), jnp.float32),
                pltpu.VMEM_SHARED((N, out_c), jnp.float32),
                pltpu.VMEM_SHARED((N, 16), jnp.float32),
                pltpu.SemaphoreType.DMA,
                pltpu.SemaphoreType.DMA,
            ],
        )
        return k(msg, dst3d, zeros, zc, ones)
    k = pl.kernel(
        _scatter_body(nbuf),
        mesh=mesh,
        compiler_params=_SC_PARAMS,
        out_type=jax.ShapeDtypeStruct((NC, N, out_c), jnp.float32),
        scratch_types=[
            pltpu.VMEM((NCH, CH), jnp.int32),
            pltpu.VMEM((nbuf, CH, out_c), jnp.float32),
            pltpu.VMEM_SHARED((N, out_c), jnp.float32),
            pltpu.SemaphoreType.DMA,
            pltpu.SemaphoreType.DMA,
        ],
    )
    return k(msg, dst3d, zeros)


def _edge_body(in_c, out_c):
    def body(ea_ref, xs_ref, wt_ref, bt_ref, msg_ref):
        eat = ea_ref[...].T
        h = jnp.dot(wt_ref[...], eat, preferred_element_type=jnp.float32)
        h = jnp.maximum(h + bt_ref[...], 0.0)
        xst = xs_ref[...].T
        acc = xst[0:1, :] * h[0:out_c, :]
        for i in range(1, in_c):
            acc = acc + xst[i:i + 1, :] * h[i * out_c:(i + 1) * out_c, :]
        msg_ref[...] = acc.T
    return body


def _edge_tc(ea, xs, wt, bt, in_c, out_c, eb=512):
    return pl.pallas_call(
        _edge_body(in_c, out_c),
        grid=(E // eb,),
        in_specs=[
            pl.BlockSpec((eb, DE), lambda i: (i, 0)),
            pl.BlockSpec((eb, in_c), lambda i: (i, 0)),
            pl.BlockSpec((in_c * out_c, DE), lambda i: (0, 0)),
            pl.BlockSpec((in_c * out_c, 1), lambda i: (0, 0)),
        ],
        out_specs=pl.BlockSpec((eb, out_c), lambda i: (i, 0)),
        out_shape=jax.ShapeDtypeStruct((E, out_c), jnp.float32),
    )(ea, xs, wt, bt)


def _update_body(x_ref, agg_ref, cnt_ref, root_ref, b_ref, h_ref):
    agg3 = agg_ref[...]
    agg = agg3[0] + agg3[1]
    cnt3 = cnt_ref[...]
    cnt = cnt3[0, :, 0:1] + cnt3[1, :, 0:1]
    inv = 1.0 / jnp.maximum(cnt, 1.0)
    h = jnp.dot(x_ref[...], root_ref[...], preferred_element_type=jnp.float32)
    h_ref[...] = jnp.maximum(h + agg * inv + b_ref[...], 0.0)


def _update_tc(x, agg, cnt, root, b, in_c, out_c):
    return pl.pallas_call(
        _update_body,
        out_shape=jax.ShapeDtypeStruct((N, out_c), jnp.float32),
    )(x, agg, cnt, root, b)


def _cbt_body(d):
    def body(ha_ref, hb_ref, o_ref):
        ha = ha_ref[...]
        hbt = hb_ref[...].T
        acc = jnp.abs(ha[:, 0:1] - hbt[0:1, :])
        for k in range(1, d):
            acc = acc + jnp.abs(ha[:, k:k + 1] - hbt[k:k + 1, :])
        o_ref[...] = acc
    return body


def _cbt_tc(h, d, ba=256, bb=256):
    return pl.pallas_call(
        _cbt_body(d),
        grid=(N // ba, N // bb),
        in_specs=[
            pl.BlockSpec((ba, d), lambda i, j: (i, 0)),
            pl.BlockSpec((bb, d), lambda i, j: (j, 0)),
        ],
        out_specs=pl.BlockSpec((ba, bb), lambda i, j: (i, j)),
        out_shape=jax.ShapeDtypeStruct((N, N), jnp.float32),
    )(h, h)


def kernel(x, edge_attr, edge_index, W_nn1, b_nn1, root1, bias1,
           W_nn2, b_nn2, root2, bias2, W_nn3, b_nn3, root3, bias3):
    src3d = edge_index[0].reshape(NW, NCH, CH)
    dst3d = edge_index[1].reshape(NW, NCH, CH)
    layers = [
        (W_nn1, b_nn1, root1, bias1, 32, 64),
        (W_nn2, b_nn2, root2, bias2, 64, 32),
        (W_nn3, b_nn3, root3, bias3, 32, 16),
    ]
    h = x
    cnt = None
    for li, (w, b, root, bias, in_c, out_c) in enumerate(layers):
        xs = _sc_gather(h, src3d, in_c)
        msg = _edge_tc(edge_attr, xs, w.T, b.reshape(-1, 1), in_c, out_c)
        if li == 0:
            agg, cnt = _sc_scatter(msg, dst3d, out_c, True)
        else:
            agg = _sc_scatter(msg, dst3d, out_c, False)
        h = _update_tc(h, agg, cnt, root, bias.reshape(1, -1), in_c, out_c)
    return _cbt_tc(h, 16)


# speedup vs baseline: 3.8202x; 1.1536x over previous
import functools

import jax
import jax.numpy as jnp
from jax import lax
from jax.experimental import pallas as pl
from jax.experimental.pallas import tpu as pltpu
from jax.experimental.pallas import tpu_sc as plsc

N = 2048
E = 65536
DE = 16
NC = 2
NS = 16
NW = NC * NS
EPT = E // NW
CH = 128
NCH = EPT // CH
NPT = N // NS


def _gather_body(nbuf):
    def body(x_hbm, src_hbm, xs_hbm, idx_v, rows_v, gsem, wsem):
        c = lax.axis_index("c")
        s = lax.axis_index("s")
        wid = s * NC + c
        pltpu.sync_copy(src_hbm.at[wid], idx_v)
        base = wid * EPT
        for w0 in range(0, NCH, nbuf):
            gs = [pltpu.async_copy(x_hbm.at[idx_v.at[w0 + t]], rows_v.at[t], gsem)
                  for t in range(nbuf)]
            for cp in gs:
                cp.wait()
            ws = [pltpu.async_copy(rows_v.at[t],
                                   xs_hbm.at[pl.ds(base + (w0 + t) * CH, CH)], wsem)
                  for t in range(nbuf)]
            for cp in ws:
                cp.wait()
    return body


_SC_PARAMS = pltpu.CompilerParams(use_tc_tiling_on_sc=False)


def _sc_gather(x, src3d, in_c):
    mesh = plsc.VectorSubcoreMesh(core_axis_name="c", subcore_axis_name="s")
    nbuf = 16 if in_c <= 32 else 8
    k = pl.kernel(
        _gather_body(nbuf),
        mesh=mesh,
        compiler_params=_SC_PARAMS,
        out_type=jax.ShapeDtypeStruct((E, in_c), jnp.float32),
        scratch_types=[
            pltpu.VMEM((NCH, CH), jnp.int32),
            pltpu.VMEM((nbuf, CH, in_c), jnp.float32),
            pltpu.SemaphoreType.DMA,
            pltpu.SemaphoreType.DMA,
        ],
    )
    return k(x, src3d)


def _scatter_cnt_body(nbuf):
    def body(msg_hbm, dst_hbm, zeros_hbm, zc_hbm, ones_hbm,
             agg_hbm, cnt_hbm, idx_v, rows_v, ones_v, agg_sh, cnt_sh,
             lsem, ssem):
        c = lax.axis_index("c")
        s = lax.axis_index("s")
        wid = s * NC + c
        r0 = s * NPT
        pltpu.sync_copy(zeros_hbm.at[pl.ds(r0, NPT)], agg_sh.at[pl.ds(r0, NPT)])
        pltpu.sync_copy(zc_hbm.at[pl.ds(r0, NPT)], cnt_sh.at[pl.ds(r0, NPT)])
        pltpu.sync_copy(ones_hbm, ones_v)
        pltpu.sync_copy(dst_hbm.at[wid], idx_v)
        plsc.subcore_barrier()
        base = wid * EPT
        for w0 in range(0, NCH, nbuf):
            ld = [pltpu.async_copy(msg_hbm.at[pl.ds(base + (w0 + t) * CH, CH)],
                                   rows_v.at[t], lsem) for t in range(nbuf)]
            for cp in ld:
                cp.wait()
            sc = [pltpu.async_copy(rows_v.at[t], agg_sh.at[idx_v.at[w0 + t]],
                                   ssem, add=True) for t in range(nbuf)]
            sc += [pltpu.async_copy(ones_v, cnt_sh.at[idx_v.at[w0 + t]],
                                    ssem, add=True) for t in range(nbuf)]
            for cp in sc:
                cp.wait()
        plsc.subcore_barrier()
        pltpu.sync_copy(agg_sh.at[pl.ds(r0, NPT)], agg_hbm.at[c, pl.ds(r0, NPT)])
        pltpu.sync_copy(cnt_sh.at[pl.ds(r0, NPT)], cnt_hbm.at[c, pl.ds(r0, NPT)])
    return body


def _scatter_body(nbuf):
    def body(msg_hbm, dst_hbm, zeros_hbm, agg_hbm, idx_v, rows_v, agg_sh,
             lsem, ssem):
        c = lax.axis_index("c")
        s = lax.axis_index("s")
        wid = s * NC + c
        r0 = s * NPT
        pltpu.sync_copy(zeros_hbm.at[pl.ds(r0, NPT)], agg_sh.at[pl.ds(r0, NPT)])
        pltpu.sync_copy(dst_hbm.at[wid], idx_v)
        plsc.subcore_barrier()
        base = wid * EPT
        for w0 in range(0, NCH, nbuf):
            ld = [pltpu.async_copy(msg_hbm.at[pl.ds(base + (w0 + t) * CH, CH)],
                                   rows_v.at[t], lsem) for t in range(nbuf)]
            for cp in ld:
                cp.wait()
            sc = [pltpu.async_copy(rows_v.at[t], agg_sh.at[idx_v.at[w0 + t]],
                                   ssem, add=True) for t in range(nbuf)]
            for cp in sc:
                cp.wait()
        plsc.subcore_barrier()
        pltpu.sync_copy(agg_sh.at[pl.ds(r0, NPT)], agg_hbm.at[c, pl.ds(r0, NPT)])
    return body


def _sc_scatter(msg, dst3d, out_c, with_cnt):
    mesh = plsc.VectorSubcoreMesh(core_axis_name="c", subcore_axis_name="s")
    zeros = jnp.zeros((N, out_c), jnp.float32)
    nbuf = 16 if out_c <= 32 else 8
    if with_cnt:
        zc = jnp.zeros((N, 16), jnp.float32)
        ones = jnp.ones((CH, 16), jnp.float32)
        k = pl.kernel(
            _scatter_cnt_body(nbuf),
            mesh=mesh,
            compiler_params=_SC_PARAMS,
            out_type=(jax.ShapeDtypeStruct((NC, N, out_c), jnp.float32),
                      jax.ShapeDtypeStruct((NC, N, 16), jnp.float32)),
            scratch_types=[
                pltpu.VMEM((NCH, CH), jnp.int32),
                pltpu.VMEM((nbuf, CH, out_c), jnp.float32),
                pltpu.VMEM((CH, 16), jnp.float32),
                pltpu.VMEM_SHARED((N, out_c), jnp.float32),
                pltpu.VMEM_SHARED((N, 16), jnp.float32),
                pltpu.SemaphoreType.DMA,
                pltpu.SemaphoreType.DMA,
            ],
        )
        return k(msg, dst3d, zeros, zc, ones)
    k = pl.kernel(
        _scatter_body(nbuf),
        mesh=mesh,
        compiler_params=_SC_PARAMS,
        out_type=jax.ShapeDtypeStruct((NC, N, out_c), jnp.float32),
        scratch_types=[
            pltpu.VMEM((NCH, CH), jnp.int32),
            pltpu.VMEM((nbuf, CH, out_c), jnp.float32),
            pltpu.VMEM_SHARED((N, out_c), jnp.float32),
            pltpu.SemaphoreType.DMA,
            pltpu.SemaphoreType.DMA,
        ],
    )
    return k(msg, dst3d, zeros)


def _edge_body(in_c, out_c):
    chi = max(1, 128 // out_c)
    def body(ea_ref, xs_ref, wt_ref, bt_ref, msg_ref):
        eat = ea_ref[...].T
        xst = xs_ref[...].T
        acc = None
        for c0 in range(0, in_c, chi):
            r0 = c0 * out_c
            hc = jnp.dot(wt_ref[r0:r0 + chi * out_c, :], eat,
                         preferred_element_type=jnp.float32)
            hc = jnp.maximum(hc + bt_ref[r0:r0 + chi * out_c, :], 0.0)
            for i in range(chi):
                term = xst[c0 + i:c0 + i + 1, :] * hc[i * out_c:(i + 1) * out_c, :]
                acc = term if acc is None else acc + term
        msg_ref[...] = acc.T
    return body


def _edge_tc(ea, xs, wt, bt, in_c, out_c, eb=1024):
    return pl.pallas_call(
        _edge_body(in_c, out_c),
        grid=(E // eb,),
        in_specs=[
            pl.BlockSpec((eb, DE), lambda i: (i, 0)),
            pl.BlockSpec((eb, in_c), lambda i: (i, 0)),
            pl.BlockSpec((in_c * out_c, DE), lambda i: (0, 0)),
            pl.BlockSpec((in_c * out_c, 1), lambda i: (0, 0)),
        ],
        out_specs=pl.BlockSpec((eb, out_c), lambda i: (i, 0)),
        out_shape=jax.ShapeDtypeStruct((E, out_c), jnp.float32),
    )(ea, xs, wt, bt)


def _update_body(x_ref, agg_ref, cnt_ref, root_ref, b_ref, h_ref):
    agg3 = agg_ref[...]
    agg = agg3[0] + agg3[1]
    cnt3 = cnt_ref[...]
    cnt = cnt3[0, :, 0:1] + cnt3[1, :, 0:1]
    inv = 1.0 / jnp.maximum(cnt, 1.0)
    h = jnp.dot(x_ref[...], root_ref[...], preferred_element_type=jnp.float32)
    h_ref[...] = jnp.maximum(h + agg * inv + b_ref[...], 0.0)


def _update_tc(x, agg, cnt, root, b, in_c, out_c):
    return pl.pallas_call(
        _update_body,
        out_shape=jax.ShapeDtypeStruct((N, out_c), jnp.float32),
    )(x, agg, cnt, root, b)


def _cbt_body(d):
    def body(ha_ref, hb_ref, o_ref):
        ha = ha_ref[...]
        hbt = hb_ref[...].T
        acc = jnp.abs(ha[:, 0:1] - hbt[0:1, :])
        for k in range(1, d):
            acc = acc + jnp.abs(ha[:, k:k + 1] - hbt[k:k + 1, :])
        o_ref[...] = acc
    return body


def _cbt_tc(h, d, ba=256, bb=256):
    return pl.pallas_call(
        _cbt_body(d),
        grid=(N // ba, N // bb),
        in_specs=[
            pl.BlockSpec((ba, d), lambda i, j: (i, 0)),
            pl.BlockSpec((bb, d), lambda i, j: (j, 0)),
        ],
        out_specs=pl.BlockSpec((ba, bb), lambda i, j: (i, j)),
        out_shape=jax.ShapeDtypeStruct((N, N), jnp.float32),
    )(h, h)


def kernel(x, edge_attr, edge_index, W_nn1, b_nn1, root1, bias1,
           W_nn2, b_nn2, root2, bias2, W_nn3, b_nn3, root3, bias3):
    src3d = edge_index[0].reshape(NW, NCH, CH)
    dst3d = edge_index[1].reshape(NW, NCH, CH)
    layers = [
        (W_nn1, b_nn1, root1, bias1, 32, 64),
        (W_nn2, b_nn2, root2, bias2, 64, 32),
        (W_nn3, b_nn3, root3, bias3, 32, 16),
    ]
    h = x
    cnt = None
    for li, (w, b, root, bias, in_c, out_c) in enumerate(layers):
        xs = _sc_gather(h, src3d, in_c)
        msg = _edge_tc(edge_attr, xs, w.T, b.reshape(-1, 1), in_c, out_c)
        if li == 0:
            agg, cnt = _sc_scatter(msg, dst3d, out_c, True)
        else:
            agg = _sc_scatter(msg, dst3d, out_c, False)
        h = _update_tc(h, agg, cnt, root, bias.reshape(1, -1), in_c, out_c)
    return _cbt_tc(h, 16)
